# baseline (speedup 1.0000x reference)
import jax
import jax.numpy as jnp
from jax import lax
from jax.experimental import pallas as pl
from jax.experimental.pallas import tpu as pltpu
from jax.experimental.pallas import tpu_sc as plsc

N_NODES = 10000
N_EDGES = 320000
D = 128

NC = 2
NS = 16
NW = NC * NS

CH = 64
EPW = N_EDGES // NW
NFC = EPW // CH
GROUPS = ((0, 40), (40, 40), (80, 40), (120, 36))
TAIL_OFF = NFC * CH
TAIL_E = EPW - TAIL_OFF

SEG = 624
TAIL_R = N_NODES - NS * SEG


def _mm_body(x_ref, w_ref, o_ref):
    o_ref[...] = lax.dot_general(
        x_ref[...], w_ref[...], (((1,), (1,)), ((), ())),
        preferred_element_type=jnp.float32)


def _matmul(x, W):
    return pl.pallas_call(
        _mm_body,
        grid=(10,),
        in_specs=[
            pl.BlockSpec((1000, D), lambda i: (i, 0)),
            pl.BlockSpec((D, D), lambda i: (0, 0)),
        ],
        out_specs=pl.BlockSpec((1000, D), lambda i: (i, 0)),
        out_shape=jax.ShapeDtypeStruct((N_NODES, D), jnp.float32),
    )(x, W)


def _agg_body(h_hbm, src_hbm, dst_hbm, out_hbm,
              src_v, dst_v, rows0, rows1, rows2, rows3,
              g0, g1, g2, g3, s0, s1, s2, s3, y_sh):
    c = lax.axis_index("c")
    s = lax.axis_index("s")
    wid = c * NS + s
    ebase = wid * EPW
    ROWS = (rows0, rows1, rows2, rows3)
    GS = (g0, g1, g2, g3)
    SS = (s0, s1, s2, s3)

    zeros16 = jnp.zeros((16,), jnp.float32)

    def zrow(i, _):
        for j in range(D // 16):
            rows0[i, pl.ds(j * 16, 16)] = zeros16
        return 0

    lax.fori_loop(0, CH, zrow, 0)

    base_row = s * SEG
    for k in range(SEG // CH):
        pltpu.sync_copy(rows0, y_sh.at[pl.ds(base_row + k * CH, CH)])
    rem = SEG % CH
    pltpu.sync_copy(rows0.at[pl.ds(0, rem)],
                    y_sh.at[pl.ds(base_row + (SEG // CH) * CH, rem)])

    @pl.when(s == NS - 1)
    def _():
        pltpu.sync_copy(rows0.at[pl.ds(0, TAIL_R)],
                        y_sh.at[pl.ds(NS * SEG, TAIL_R)])

    plsc.subcore_barrier()

    def gat(lc, b):
        return pltpu.make_async_copy(
            h_hbm.at[src_v.at[pl.ds(lc * CH, CH)]], ROWS[b], GS[b])

    def sca(lc, b):
        return pltpu.make_async_copy(
            ROWS[b], y_sh.at[dst_v.at[pl.ds(lc * CH, CH)]], SS[b])

    for goff, nfull in GROUPS:
        stage = nfull * CH + (TAIL_E if goff == 120 else 0)
        pltpu.sync_copy(src_hbm.at[pl.ds(ebase + goff * CH, stage)],
                        src_v.at[pl.ds(0, stage)])
        pltpu.sync_copy(dst_hbm.at[pl.ds(ebase + goff * CH, stage)],
                        dst_v.at[pl.ds(0, stage)])

        gat(0, 0).start()
        gat(1, 1).start()

        def body(g4, _):
            for b in range(4):
                lc = g4 * 4 + b
                gat(lc, b).wait()
                sca(lc, b).start(add=True)
                b2 = (b + 2) % 4

                @pl.when(lc >= 2)
                def _():
                    sca(lc - 2, b2).wait()

                @pl.when(lc + 2 < nfull)
                def _():
                    gat(lc + 2, b2).start()

            return 0

        lax.fori_loop(0, nfull // 4, body, 0)

        sca(nfull - 2, (nfull - 2) % 4).wait()
        sca(nfull - 1, (nfull - 1) % 4).wait()

    toff = TAIL_OFF - GROUPS[3][0] * CH
    pltpu.sync_copy(h_hbm.at[src_v.at[pl.ds(toff, TAIL_E)]],
                    rows0.at[pl.ds(0, TAIL_E)])
    pltpu.sync_copy(rows0.at[pl.ds(0, TAIL_E)],
                    y_sh.at[dst_v.at[pl.ds(toff, TAIL_E)]], add=True)

    plsc.subcore_barrier()

    pltpu.sync_copy(y_sh.at[pl.ds(base_row, SEG)],
                    out_hbm.at[c, pl.ds(base_row, SEG)])

    @pl.when(s == NS - 1)
    def _():
        pltpu.sync_copy(y_sh.at[pl.ds(NS * SEG, TAIL_R)],
                        out_hbm.at[c, pl.ds(NS * SEG, TAIL_R)])


def _aggregate(h, src, dst):
    mesh = plsc.VectorSubcoreMesh(
        core_axis_name="c", subcore_axis_name="s", num_cores=NC,
        num_subcores=NS)
    f = pl.kernel(
        _agg_body,
        out_type=jax.ShapeDtypeStruct((NC, N_NODES, D), jnp.float32),
        mesh=mesh,
        scratch_types=[
            pltpu.VMEM((2560 + 16,), jnp.int32),
            pltpu.VMEM((2560 + 16,), jnp.int32),
            pltpu.VMEM((CH, D), jnp.float32),
            pltpu.VMEM((CH, D), jnp.float32),
            pltpu.VMEM((CH, D), jnp.float32),
            pltpu.VMEM((CH, D), jnp.float32),
            pltpu.SemaphoreType.DMA,
            pltpu.SemaphoreType.DMA,
            pltpu.SemaphoreType.DMA,
            pltpu.SemaphoreType.DMA,
            pltpu.SemaphoreType.DMA,
            pltpu.SemaphoreType.DMA,
            pltpu.SemaphoreType.DMA,
            pltpu.SemaphoreType.DMA,
            pltpu.VMEM_SHARED((N_NODES, D), jnp.float32),
        ],
    )
    return f(h, src, dst)


def _add_body(a_ref, o_ref):
    o_ref[...] = a_ref[0] + a_ref[1]


def _combine(reps):
    return pl.pallas_call(
        _add_body,
        grid=(10,),
        in_specs=[pl.BlockSpec((NC, 1000, D), lambda i: (0, i, 0))],
        out_specs=pl.BlockSpec((1000, D), lambda i: (i, 0)),
        out_shape=jax.ShapeDtypeStruct((N_NODES, D), jnp.float32),
    )(reps)


def kernel(x, edge_index, W):
    h = _matmul(x, W)
    reps = _aggregate(h, edge_index[0], edge_index[1])
    return _combine(reps)
